# baseline (speedup 1.0000x reference)
import functools

import jax
import jax.numpy as jnp
from jax.experimental import pallas as pl
from jax.experimental.pallas import tpu as pltpu

T, D, E, K, F = 2048, 1024, 8, 2, 1024
BT = 256
NB = (K * T) // BT + (E - 1)
P = NB * BT


def _router_body(x_ref, wg_ref, topi_ref, topv_ref):
    logits = jnp.dot(x_ref[...], wg_ref[...], preferred_element_type=jnp.float32)
    e_idx = jax.lax.broadcasted_iota(jnp.int32, logits.shape, 1)
    m1 = jnp.max(logits, axis=-1, keepdims=True)
    i1 = jnp.min(jnp.where(logits == m1, e_idx, E), axis=-1, keepdims=True)
    rest = jnp.where(e_idx == i1, -jnp.inf, logits)
    m2 = jnp.max(rest, axis=-1, keepdims=True)
    i2 = jnp.min(jnp.where(rest == m2, e_idx, E), axis=-1, keepdims=True)
    w1 = 1.0 / (1.0 + jnp.exp(m2 - m1))
    topi_ref[...] = jnp.concatenate([i1, i2], axis=1)
    topv_ref[...] = jnp.concatenate([w1, 1.0 - w1], axis=1)


def _router(x, Wg):
    return pl.pallas_call(
        _router_body,
        out_shape=(
            jax.ShapeDtypeStruct((T, K), jnp.int32),
            jax.ShapeDtypeStruct((T, K), jnp.float32),
        ),
    )(x, Wg)


def _moe_body(be_ref, x_ref, st_ref, wg_ref, wu_ref, wd_ref, ws_ref, out_ref,
              wg16, wu16, wd16):
    i = pl.program_id(0)

    @pl.when(i == 0)
    def _():
        out_ref[...] = jnp.zeros_like(out_ref)

    @pl.when(i < be_ref[NB])
    def _():
        @pl.when(jnp.logical_or(i == 0, be_ref[i] != be_ref[jnp.maximum(i - 1, 0)]))
        def _():
            wg16[...] = wg_ref[0].astype(jnp.bfloat16)
            wu16[...] = wu_ref[0].astype(jnp.bfloat16)
            wd16[...] = wd_ref[0].astype(jnp.bfloat16)

        tok_iota = jax.lax.broadcasted_iota(jnp.int32, (BT, T), 1)
        onehot = (tok_iota == st_ref[0]).astype(jnp.bfloat16)
        xb = jnp.dot(onehot, x_ref[...],
                     preferred_element_type=jnp.float32
                     ).astype(jnp.bfloat16)
        g = jnp.dot(xb, wg16[...], preferred_element_type=jnp.float32)
        u = jnp.dot(xb, wu16[...], preferred_element_type=jnp.float32)
        h = ((g * jax.nn.sigmoid(g)) * u).astype(jnp.bfloat16)
        y = jnp.dot(h, wd16[...], preferred_element_type=jnp.float32)
        ysc = (y * ws_ref[0]).astype(jnp.bfloat16)
        out_ref[...] += jax.lax.dot_general(
            onehot, ysc, (((0,), (0,)), ((), ())),
            preferred_element_type=jnp.float32)


def _moe_fused(block_expert, x16, sort_tok, w_gate, w_up, w_down, sort_w):
    grid_spec = pltpu.PrefetchScalarGridSpec(
        num_scalar_prefetch=1,
        grid=(NB,),
        in_specs=[
            pl.BlockSpec((T, D), lambda i, be: (0, 0)),
            pl.BlockSpec((1, BT, 1), lambda i, be: (i, 0, 0)),
            pl.BlockSpec((1, D, F), lambda i, be: (be[i], 0, 0)),
            pl.BlockSpec((1, D, F), lambda i, be: (be[i], 0, 0)),
            pl.BlockSpec((1, F, D), lambda i, be: (be[i], 0, 0)),
            pl.BlockSpec((1, BT, 1), lambda i, be: (i, 0, 0)),
        ],
        out_specs=pl.BlockSpec((T, D), lambda i, be: (0, 0)),
        scratch_shapes=[
            pltpu.VMEM((D, F), jnp.bfloat16),
            pltpu.VMEM((D, F), jnp.bfloat16),
            pltpu.VMEM((F, D), jnp.bfloat16),
        ],
    )
    return pl.pallas_call(
        _moe_body,
        grid_spec=grid_spec,
        out_shape=jax.ShapeDtypeStruct((T, D), jnp.float32),
    )(block_expert, x16, sort_tok.reshape(NB, BT, 1), w_gate, w_up, w_down,
      sort_w.reshape(NB, BT, 1))


def kernel(x, Wg, w_gate, w_up, w_down):
    topi, topv = _router(x, Wg)

    ef = topi.reshape(-1)
    oh = (ef[:, None] == jnp.arange(E, dtype=jnp.int32)[None, :]).astype(jnp.int32)
    rank = jnp.sum((jnp.cumsum(oh, axis=0) - oh) * oh, axis=1)
    counts = jnp.sum(oh, axis=0)
    counts_pad = ((counts + BT - 1) // BT) * BT
    cum_pad = jnp.cumsum(counts_pad)
    seg_off = cum_pad - counts_pad
    pos = (seg_off[ef] + rank).astype(jnp.int32)
    tok = (jnp.arange(K * T, dtype=jnp.int32) // K)
    sort_tok = jnp.zeros((P,), jnp.int32).at[pos].set(tok)
    sort_w = jnp.zeros((P,), jnp.float32).at[pos].set(topv.reshape(-1))
    blk_start = jnp.arange(NB, dtype=jnp.int32) * BT
    nreal = (cum_pad[E - 1] // BT).astype(jnp.int32)
    raw_be = jnp.minimum(
        jnp.searchsorted(cum_pad, blk_start, side="right"), E - 1
    ).astype(jnp.int32)
    last_e = raw_be[nreal - 1]
    block_expert = jnp.concatenate([
        jnp.where(jnp.arange(NB, dtype=jnp.int32) < nreal, raw_be, last_e),
        nreal[None],
    ])

    return _moe_fused(block_expert, x.astype(jnp.bfloat16), sort_tok,
                      w_gate, w_up, w_down, sort_w)


# speedup vs baseline: 1.2958x; 1.2958x over previous
import functools

import jax
import jax.numpy as jnp
from jax import lax
from jax.experimental import pallas as pl
from jax.experimental.pallas import tpu as pltpu
from jax.experimental.pallas import tpu_sc as plsc

T, D, E, K, F = 2048, 1024, 8, 2, 1024
BT = 256
NB = (K * T) // BT + (E - 1)
P = NB * BT
NP = K * T
RB = 128
NR = NP // RB


def _router_body(x_ref, wg_ref, pos_ref, w_ref, be_ref):
    logits = jnp.dot(x_ref[...], wg_ref[...], preferred_element_type=jnp.float32)
    e_idx = jax.lax.broadcasted_iota(jnp.int32, logits.shape, 1)
    m1 = jnp.max(logits, axis=-1, keepdims=True)
    i1 = jnp.min(jnp.where(logits == m1, e_idx, E), axis=-1, keepdims=True)
    rest = jnp.where(e_idx == i1, -jnp.inf, logits)
    m2 = jnp.max(rest, axis=-1, keepdims=True)
    i2 = jnp.min(jnp.where(rest == m2, e_idx, E), axis=-1, keepdims=True)
    w1 = 1.0 / (1.0 + jnp.exp(m2 - m1))
    oh1 = (e_idx == i1).astype(jnp.bfloat16)
    oh2 = (e_idx == i2).astype(jnp.bfloat16)

    r_iota = jax.lax.broadcasted_iota(jnp.int32, (RB, RB), 0)
    c_iota = jax.lax.broadcasted_iota(jnp.int32, (RB, RB), 1)
    ltri = (c_iota < r_iota).astype(jnp.bfloat16)
    ones_rb = jnp.ones((1, RB), jnp.bfloat16)

    ranks = []
    tots = []
    for r in range(NR):
        ohr = (oh1 if r < NR // 2 else oh2)[
            (r % (NR // 2)) * RB:(r % (NR // 2)) * RB + RB]
        ranks.append(jnp.dot(ltri, ohr, preferred_element_type=jnp.float32))
        tots.append(jnp.dot(ones_rb, ohr, preferred_element_type=jnp.float32))
    tot = jnp.concatenate(tots, axis=0)

    rb_iota = jax.lax.broadcasted_iota(jnp.int32, (NR, NR), 0)
    cb_iota = jax.lax.broadcasted_iota(jnp.int32, (NR, NR), 1)
    ltri_b = (cb_iota < rb_iota).astype(jnp.bfloat16)
    excl_b = jnp.dot(ltri_b, tot.astype(jnp.bfloat16),
                     preferred_element_type=jnp.float32)
    counts = jnp.sum(tot, axis=0, keepdims=True)

    counts_pad = jnp.floor((counts + (BT - 1)) / BT) * BT
    ei = jax.lax.broadcasted_iota(jnp.int32, (E, E), 0)
    ej = jax.lax.broadcasted_iota(jnp.int32, (E, E), 1)
    utri_inc = (ei <= ej).astype(jnp.bfloat16)
    cum_pad = jnp.dot(counts_pad.astype(jnp.bfloat16) / 256.0, utri_inc,
                      preferred_element_type=jnp.float32) * 256.0
    seg_off = cum_pad - counts_pad

    for r in range(NR):
        ohr = (oh1 if r < NR // 2 else oh2)[
            (r % (NR // 2)) * RB:(r % (NR // 2)) * RB + RB].astype(jnp.float32)
        pos_r = (ranks[r] + excl_b[r:r + 1] + seg_off) * ohr
        pos_flat = jnp.sum(pos_r, axis=1).astype(jnp.int32)
        pos_ref[r:r + 1, :] = pos_flat.reshape(1, RB)

    w_ref[0:NR // 2, :] = w1.reshape(NR // 2, RB)
    w_ref[NR // 2:NR, :] = (1.0 - w1).reshape(NR // 2, RB)

    blk = jax.lax.broadcasted_iota(jnp.int32, (1, 32), 1)
    raw = jnp.zeros((1, 32), jnp.int32)
    for e in range(E):
        raw = raw + (blk * BT >= cum_pad[0, e].astype(jnp.int32)).astype(jnp.int32)
    nreal = (cum_pad[0, E - 1] / BT).astype(jnp.int32)
    last_cnt = jnp.int32(0)
    for e in range(E):
        last_cnt = last_cnt + (
            cum_pad[0, e].astype(jnp.int32) <= (nreal - 1) * BT).astype(jnp.int32)
    last_e = jnp.minimum(last_cnt, E - 1)
    isreal = (blk < nreal).astype(jnp.int32)
    val = jnp.minimum(raw, E - 1) * isreal + last_e * (1 - isreal)
    isnb = (blk == NB).astype(jnp.int32)
    be_ref[...] = nreal * isnb + val * (1 - isnb)


def _router(x, Wg):
    return pl.pallas_call(
        _router_body,
        out_shape=(
            jax.ShapeDtypeStruct((NR, RB), jnp.int32),
            jax.ShapeDtypeStruct((NR, RB), jnp.float32),
            jax.ShapeDtypeStruct((1, 32), jnp.int32),
        ),
    )(x, Wg)


W = 16
CPW = NP // W
PW = P // W


def _scatter_body(pos_hbm, w_hbm, st_hbm, sw_hbm,
                  pos0, pos1, w0, w1, tok0, tok1, zb_i, zb_f,
                  st_sh, sw_sh):
    cid = lax.axis_index("c")
    wid = lax.axis_index("s")
    lanes = lax.broadcasted_iota(jnp.int32, (16,), 0)
    base = wid * CPW

    pltpu.sync_copy(pos_hbm.at[pl.ds(base, RB)], pos0)
    pltpu.sync_copy(pos_hbm.at[pl.ds(base + RB, RB)], pos1)
    pltpu.sync_copy(w_hbm.at[pl.ds(base, RB)], w0)
    pltpu.sync_copy(w_hbm.at[pl.ds(base + RB, RB)], w1)

    for c in range(RB // 16):
        tok0[pl.ds(16 * c, 16)] = (base + 16 * c + lanes) & (T - 1)
        tok1[pl.ds(16 * c, 16)] = (base + RB + 16 * c + lanes) & (T - 1)

    for c in range(PW // 16):
        zb_i[pl.ds(16 * c, 16)] = jnp.zeros((16,), jnp.int32)
        zb_f[pl.ds(16 * c, 16)] = jnp.zeros((16,), jnp.float32)
    pltpu.sync_copy(zb_i, st_sh.at[pl.ds(wid * PW, PW)])
    pltpu.sync_copy(zb_f, sw_sh.at[pl.ds(wid * PW, PW)])
    plsc.subcore_barrier()

    pltpu.sync_copy(tok0, st_sh.at[pos0])
    pltpu.sync_copy(tok1, st_sh.at[pos1])
    pltpu.sync_copy(w0, sw_sh.at[pos0])
    pltpu.sync_copy(w1, sw_sh.at[pos1])
    plsc.subcore_barrier()

    @pl.when(cid == 0)
    def _():
        pltpu.sync_copy(st_sh.at[pl.ds(wid * PW, PW)], zb_i)
        pltpu.sync_copy(zb_i, st_hbm.at[pl.ds(wid * PW, PW)])
        pltpu.sync_copy(sw_sh.at[pl.ds(wid * PW, PW)], zb_f)
        pltpu.sync_copy(zb_f, sw_hbm.at[pl.ds(wid * PW, PW)])


@functools.cache
def _make_scatter():
    @functools.partial(
        pl.kernel,
        out_type=(
            jax.ShapeDtypeStruct((P,), jnp.int32),
            jax.ShapeDtypeStruct((P,), jnp.float32),
        ),
        mesh=plsc.VectorSubcoreMesh(core_axis_name="c", subcore_axis_name="s"),
        scratch_types=[
            pltpu.VMEM((RB,), jnp.int32),
            pltpu.VMEM((RB,), jnp.int32),
            pltpu.VMEM((RB,), jnp.float32),
            pltpu.VMEM((RB,), jnp.float32),
            pltpu.VMEM((RB,), jnp.int32),
            pltpu.VMEM((RB,), jnp.int32),
            pltpu.VMEM((PW,), jnp.int32),
            pltpu.VMEM((PW,), jnp.float32),
            pltpu.VMEM_SHARED((P,), jnp.int32),
            pltpu.VMEM_SHARED((P,), jnp.float32),
        ],
    )
    def _scatter_kernel(pos_hbm, w_hbm, st_hbm, sw_hbm, *scratch):
        _scatter_body(pos_hbm, w_hbm, st_hbm, sw_hbm, *scratch)

    return _scatter_kernel


def _scatter(pos, wflat):
    return _make_scatter()(pos.reshape(NP), wflat.reshape(NP))


def _moe_body(be_ref, x_ref, st_ref, wg_ref, wu_ref, wd_ref, ws_ref, out_ref):
    i = pl.program_id(0)

    @pl.when(i == 0)
    def _():
        out_ref[...] = jnp.zeros_like(out_ref)

    @pl.when(i < be_ref[NB])
    def _():
        tok_iota = jax.lax.broadcasted_iota(jnp.int32, (BT, T), 1)
        onehot = (tok_iota == st_ref[0]).astype(jnp.bfloat16)
        xb = jnp.dot(onehot, x_ref[...],
                     preferred_element_type=jnp.float32
                     ).astype(jnp.bfloat16)
        wg = wg_ref[0].astype(jnp.bfloat16)
        wu = wu_ref[0].astype(jnp.bfloat16)
        wd = wd_ref[0].astype(jnp.bfloat16)
        g = jnp.dot(xb, wg, preferred_element_type=jnp.float32)
        u = jnp.dot(xb, wu, preferred_element_type=jnp.float32)
        h = ((g * jax.nn.sigmoid(g)) * u).astype(jnp.bfloat16)
        y = jnp.dot(h, wd, preferred_element_type=jnp.float32)
        ysc = (y * ws_ref[0]).astype(jnp.bfloat16)
        out_ref[...] += jax.lax.dot_general(
            onehot, ysc, (((0,), (0,)), ((), ())),
            preferred_element_type=jnp.float32)


def _moe_fused(block_expert, x16, sort_tok, w_gate, w_up, w_down, sort_w):
    grid_spec = pltpu.PrefetchScalarGridSpec(
        num_scalar_prefetch=1,
        grid=(NB,),
        in_specs=[
            pl.BlockSpec((T, D), lambda i, be: (0, 0)),
            pl.BlockSpec((1, BT, 1), lambda i, be: (i, 0, 0)),
            pl.BlockSpec((1, D, F), lambda i, be: (be[i], 0, 0)),
            pl.BlockSpec((1, D, F), lambda i, be: (be[i], 0, 0)),
            pl.BlockSpec((1, F, D), lambda i, be: (be[i], 0, 0)),
            pl.BlockSpec((1, BT, 1), lambda i, be: (i, 0, 0)),
        ],
        out_specs=pl.BlockSpec((T, D), lambda i, be: (0, 0)),
    )
    return pl.pallas_call(
        _moe_body,
        grid_spec=grid_spec,
        out_shape=jax.ShapeDtypeStruct((T, D), jnp.float32),
    )(block_expert, x16, sort_tok.reshape(NB, BT, 1), w_gate, w_up, w_down,
      sort_w.reshape(NB, BT, 1))


def kernel(x, Wg, w_gate, w_up, w_down):
    pos, wflat, be2 = _router(x, Wg)
    sort_tok, sort_w = _scatter(pos, wflat)
    block_expert = be2[0, :NB + 1]

    return _moe_fused(block_expert, x.astype(jnp.bfloat16), sort_tok,
                      w_gate, w_up, w_down, sort_w)


# speedup vs baseline: 1.3566x; 1.0469x over previous
import functools

import jax
import jax.numpy as jnp
from jax import lax
from jax.experimental import pallas as pl
from jax.experimental.pallas import tpu as pltpu
from jax.experimental.pallas import tpu_sc as plsc

T, D, E, K, F = 2048, 1024, 8, 2, 1024
BT = 256
NB = (K * T) // BT + (E - 1)
P = NB * BT
NP = K * T
RB = 128
NR = NP // RB


def _router_body(x_ref, wg_ref, pos_ref, w_ref, be_ref):
    logits = jnp.dot(x_ref[...], wg_ref[...], preferred_element_type=jnp.float32)
    e_idx = jax.lax.broadcasted_iota(jnp.int32, logits.shape, 1)
    m1 = jnp.max(logits, axis=-1, keepdims=True)
    i1 = jnp.min(jnp.where(logits == m1, e_idx, E), axis=-1, keepdims=True)
    rest = jnp.where(e_idx == i1, -jnp.inf, logits)
    m2 = jnp.max(rest, axis=-1, keepdims=True)
    i2 = jnp.min(jnp.where(rest == m2, e_idx, E), axis=-1, keepdims=True)
    w1 = 1.0 / (1.0 + jnp.exp(m2 - m1))
    oh1 = (e_idx == i1).astype(jnp.bfloat16)
    oh2 = (e_idx == i2).astype(jnp.bfloat16)

    r_iota = jax.lax.broadcasted_iota(jnp.int32, (RB, RB), 0)
    c_iota = jax.lax.broadcasted_iota(jnp.int32, (RB, RB), 1)
    ltri = (c_iota < r_iota).astype(jnp.bfloat16)
    ones_rb = jnp.ones((1, RB), jnp.bfloat16)

    ranks = []
    tots = []
    for r in range(NR):
        ohr = (oh1 if r < NR // 2 else oh2)[
            (r % (NR // 2)) * RB:(r % (NR // 2)) * RB + RB]
        ranks.append(jnp.dot(ltri, ohr, preferred_element_type=jnp.float32))
        tots.append(jnp.dot(ones_rb, ohr, preferred_element_type=jnp.float32))
    tot = jnp.concatenate(tots, axis=0)

    rb_iota = jax.lax.broadcasted_iota(jnp.int32, (NR, NR), 0)
    cb_iota = jax.lax.broadcasted_iota(jnp.int32, (NR, NR), 1)
    ltri_b = (cb_iota < rb_iota).astype(jnp.bfloat16)
    excl_b = jnp.dot(ltri_b, tot.astype(jnp.bfloat16),
                     preferred_element_type=jnp.float32)
    counts = jnp.sum(tot, axis=0, keepdims=True)

    counts_pad = jnp.floor((counts + (BT - 1)) / BT) * BT
    ei = jax.lax.broadcasted_iota(jnp.int32, (E, E), 0)
    ej = jax.lax.broadcasted_iota(jnp.int32, (E, E), 1)
    utri_inc = (ei <= ej).astype(jnp.bfloat16)
    cum_pad = jnp.dot(counts_pad.astype(jnp.bfloat16) / 256.0, utri_inc,
                      preferred_element_type=jnp.float32) * 256.0
    seg_off = cum_pad - counts_pad

    for r in range(NR):
        ohr = (oh1 if r < NR // 2 else oh2)[
            (r % (NR // 2)) * RB:(r % (NR // 2)) * RB + RB].astype(jnp.float32)
        pos_r = (ranks[r] + excl_b[r:r + 1] + seg_off) * ohr
        pos_flat = jnp.sum(pos_r, axis=1).astype(jnp.int32)
        pos_ref[r:r + 1, :] = pos_flat.reshape(1, RB)

    w_ref[0:NR // 2, :] = w1.reshape(NR // 2, RB)
    w_ref[NR // 2:NR, :] = (1.0 - w1).reshape(NR // 2, RB)

    blk = jax.lax.broadcasted_iota(jnp.int32, (1, 32), 1)
    raw = jnp.zeros((1, 32), jnp.int32)
    for e in range(E):
        raw = raw + (blk * BT >= cum_pad[0, e].astype(jnp.int32)).astype(jnp.int32)
    nreal = (cum_pad[0, E - 1] / BT).astype(jnp.int32)
    last_cnt = jnp.int32(0)
    for e in range(E):
        last_cnt = last_cnt + (
            cum_pad[0, e].astype(jnp.int32) <= (nreal - 1) * BT).astype(jnp.int32)
    last_e = jnp.minimum(last_cnt, E - 1)
    isreal = (blk < nreal).astype(jnp.int32)
    val = jnp.minimum(raw, E - 1) * isreal + last_e * (1 - isreal)
    isnb = (blk == NB).astype(jnp.int32)
    be_ref[...] = nreal * isnb + val * (1 - isnb)


def _router(x, Wg):
    return pl.pallas_call(
        _router_body,
        out_shape=(
            jax.ShapeDtypeStruct((NR, RB), jnp.int32),
            jax.ShapeDtypeStruct((NR, RB), jnp.float32),
            jax.ShapeDtypeStruct((1, 32), jnp.int32),
        ),
    )(x, Wg)


W = 16
CPW = NP // W
PW = P // W


def _scatter_body(pos_hbm, w_hbm, st_hbm, sw_hbm,
                  pos0, pos1, w0, w1, tok0, tok1, zb_i, zb_f,
                  st_sh, sw_sh):
    cid = lax.axis_index("c")
    wid = lax.axis_index("s")
    lanes = lax.broadcasted_iota(jnp.int32, (16,), 0)
    base = wid * CPW

    pltpu.sync_copy(pos_hbm.at[pl.ds(base, RB)], pos0)
    pltpu.sync_copy(pos_hbm.at[pl.ds(base + RB, RB)], pos1)
    pltpu.sync_copy(w_hbm.at[pl.ds(base, RB)], w0)
    pltpu.sync_copy(w_hbm.at[pl.ds(base + RB, RB)], w1)

    for c in range(RB // 16):
        tok0[pl.ds(16 * c, 16)] = (base + 16 * c + lanes) & (T - 1)
        tok1[pl.ds(16 * c, 16)] = (base + RB + 16 * c + lanes) & (T - 1)

    for c in range(PW // 16):
        zb_i[pl.ds(16 * c, 16)] = jnp.zeros((16,), jnp.int32)
        zb_f[pl.ds(16 * c, 16)] = jnp.zeros((16,), jnp.float32)
    pltpu.sync_copy(zb_i, st_sh.at[pl.ds(wid * PW, PW)])
    pltpu.sync_copy(zb_f, sw_sh.at[pl.ds(wid * PW, PW)])
    plsc.subcore_barrier()

    pltpu.sync_copy(tok0, st_sh.at[pos0])
    pltpu.sync_copy(tok1, st_sh.at[pos1])
    pltpu.sync_copy(w0, sw_sh.at[pos0])
    pltpu.sync_copy(w1, sw_sh.at[pos1])
    plsc.subcore_barrier()

    @pl.when(cid == 0)
    def _():
        pltpu.sync_copy(st_sh.at[pl.ds(wid * PW, PW)], zb_i)
        pltpu.sync_copy(zb_i, st_hbm.at[pl.ds(wid * PW, PW)])
        pltpu.sync_copy(sw_sh.at[pl.ds(wid * PW, PW)], zb_f)
        pltpu.sync_copy(zb_f, sw_hbm.at[pl.ds(wid * PW, PW)])


@functools.cache
def _make_scatter():
    @functools.partial(
        pl.kernel,
        out_type=(
            jax.ShapeDtypeStruct((P,), jnp.int32),
            jax.ShapeDtypeStruct((P,), jnp.float32),
        ),
        mesh=plsc.VectorSubcoreMesh(core_axis_name="c", subcore_axis_name="s"),
        scratch_types=[
            pltpu.VMEM((RB,), jnp.int32),
            pltpu.VMEM((RB,), jnp.int32),
            pltpu.VMEM((RB,), jnp.float32),
            pltpu.VMEM((RB,), jnp.float32),
            pltpu.VMEM((RB,), jnp.int32),
            pltpu.VMEM((RB,), jnp.int32),
            pltpu.VMEM((PW,), jnp.int32),
            pltpu.VMEM((PW,), jnp.float32),
            pltpu.VMEM_SHARED((P,), jnp.int32),
            pltpu.VMEM_SHARED((P,), jnp.float32),
        ],
    )
    def _scatter_kernel(pos_hbm, w_hbm, st_hbm, sw_hbm, *scratch):
        _scatter_body(pos_hbm, w_hbm, st_hbm, sw_hbm, *scratch)

    return _scatter_kernel


def _scatter(pos, wflat):
    return _make_scatter()(pos.reshape(NP), wflat.reshape(NP))


def _moe_body(be_ref, x_ref, st_ref, wg_ref, wu_ref, wd_ref, ws_ref, out_ref):
    i = pl.program_id(0)

    @pl.when(i == 0)
    def _():
        out_ref[...] = jnp.zeros_like(out_ref)

    @pl.when(i < be_ref[NB])
    def _():
        tok_iota = jax.lax.broadcasted_iota(jnp.int32, (BT, T), 1)
        st_col = st_ref[0, 0].reshape(BT, 1)
        onehot = (tok_iota == st_col).astype(jnp.bfloat16)
        xb = jnp.dot(onehot, x_ref[...],
                     preferred_element_type=jnp.float32
                     ).astype(jnp.bfloat16)
        wg = wg_ref[0].astype(jnp.bfloat16)
        wu = wu_ref[0].astype(jnp.bfloat16)
        wd = wd_ref[0].astype(jnp.bfloat16)
        g = jnp.dot(xb, wg, preferred_element_type=jnp.float32)
        u = jnp.dot(xb, wu, preferred_element_type=jnp.float32)
        h = ((g * jax.nn.sigmoid(g)) * u).astype(jnp.bfloat16)
        y = jnp.dot(h, wd, preferred_element_type=jnp.float32)
        ysc = (y * ws_ref[0, 0].reshape(BT, 1)).astype(jnp.bfloat16)
        out_ref[...] += jax.lax.dot_general(
            onehot, ysc, (((0,), (0,)), ((), ())),
            preferred_element_type=jnp.float32)


def _moe_fused(block_expert, x16, sort_tok, w_gate, w_up, w_down, sort_w):
    grid_spec = pltpu.PrefetchScalarGridSpec(
        num_scalar_prefetch=1,
        grid=(NB,),
        in_specs=[
            pl.BlockSpec((T, D), lambda i, be: (0, 0)),
            pl.BlockSpec((1, 1, BT), lambda i, be: (i, 0, 0)),
            pl.BlockSpec((1, D, F), lambda i, be: (be[i], 0, 0)),
            pl.BlockSpec((1, D, F), lambda i, be: (be[i], 0, 0)),
            pl.BlockSpec((1, F, D), lambda i, be: (be[i], 0, 0)),
            pl.BlockSpec((1, 1, BT), lambda i, be: (i, 0, 0)),
        ],
        out_specs=pl.BlockSpec((T, D), lambda i, be: (0, 0)),
    )
    return pl.pallas_call(
        _moe_body,
        grid_spec=grid_spec,
        out_shape=jax.ShapeDtypeStruct((T, D), jnp.float32),
    )(block_expert, x16, sort_tok.reshape(NB, 1, BT), w_gate, w_up, w_down,
      sort_w.reshape(NB, 1, BT))


def kernel(x, Wg, w_gate, w_up, w_down):
    pos, wflat, be2 = _router(x, Wg)
    sort_tok, sort_w = _scatter(pos, wflat)
    block_expert = be2[0, :NB + 1]

    return _moe_fused(block_expert, x.astype(jnp.bfloat16), sort_tok,
                      w_gate, w_up, w_down, sort_w)


# speedup vs baseline: 1.3645x; 1.0058x over previous
import functools

import jax
import jax.numpy as jnp
from jax import lax
from jax.experimental import pallas as pl
from jax.experimental.pallas import tpu as pltpu
from jax.experimental.pallas import tpu_sc as plsc

T, D, E, K, F = 2048, 1024, 8, 2, 1024
BT = 256
NB = (K * T) // BT + (E - 1)
P = NB * BT
NP = K * T
RB = 128
NR = NP // RB


def _router_body(x_ref, wg_ref, pos_ref, w_ref, be_ref, x16_ref):
    x16_ref[...] = x_ref[...].astype(jnp.bfloat16)
    logits = jnp.dot(x_ref[...], wg_ref[...], preferred_element_type=jnp.float32)
    e_idx = jax.lax.broadcasted_iota(jnp.int32, logits.shape, 1)
    m1 = jnp.max(logits, axis=-1, keepdims=True)
    i1 = jnp.min(jnp.where(logits == m1, e_idx, E), axis=-1, keepdims=True)
    rest = jnp.where(e_idx == i1, -jnp.inf, logits)
    m2 = jnp.max(rest, axis=-1, keepdims=True)
    i2 = jnp.min(jnp.where(rest == m2, e_idx, E), axis=-1, keepdims=True)
    w1 = 1.0 / (1.0 + jnp.exp(m2 - m1))
    oh1 = (e_idx == i1).astype(jnp.bfloat16)
    oh2 = (e_idx == i2).astype(jnp.bfloat16)

    r_iota = jax.lax.broadcasted_iota(jnp.int32, (RB, RB), 0)
    c_iota = jax.lax.broadcasted_iota(jnp.int32, (RB, RB), 1)
    ltri = (c_iota < r_iota).astype(jnp.bfloat16)
    ones_rb = jnp.ones((1, RB), jnp.bfloat16)

    ranks = []
    tots = []
    for r in range(NR):
        ohr = (oh1 if r < NR // 2 else oh2)[
            (r % (NR // 2)) * RB:(r % (NR // 2)) * RB + RB]
        ranks.append(jnp.dot(ltri, ohr, preferred_element_type=jnp.float32))
        tots.append(jnp.dot(ones_rb, ohr, preferred_element_type=jnp.float32))
    tot = jnp.concatenate(tots, axis=0)

    rb_iota = jax.lax.broadcasted_iota(jnp.int32, (NR, NR), 0)
    cb_iota = jax.lax.broadcasted_iota(jnp.int32, (NR, NR), 1)
    ltri_b = (cb_iota < rb_iota).astype(jnp.bfloat16)
    excl_b = jnp.dot(ltri_b, tot.astype(jnp.bfloat16),
                     preferred_element_type=jnp.float32)
    counts = jnp.sum(tot, axis=0, keepdims=True)

    counts_pad = jnp.floor((counts + (BT - 1)) / BT) * BT
    ei = jax.lax.broadcasted_iota(jnp.int32, (E, E), 0)
    ej = jax.lax.broadcasted_iota(jnp.int32, (E, E), 1)
    utri_inc = (ei <= ej).astype(jnp.bfloat16)
    cum_pad = jnp.dot(counts_pad.astype(jnp.bfloat16) / 256.0, utri_inc,
                      preferred_element_type=jnp.float32) * 256.0
    seg_off = cum_pad - counts_pad

    for r in range(NR):
        ohr = (oh1 if r < NR // 2 else oh2)[
            (r % (NR // 2)) * RB:(r % (NR // 2)) * RB + RB].astype(jnp.float32)
        pos_r = (ranks[r] + excl_b[r:r + 1] + seg_off) * ohr
        pos_flat = jnp.sum(pos_r, axis=1).astype(jnp.int32)
        pos_ref[r:r + 1, :] = pos_flat.reshape(1, RB)

    w_ref[0:NR // 2, :] = w1.reshape(NR // 2, RB)
    w_ref[NR // 2:NR, :] = (1.0 - w1).reshape(NR // 2, RB)

    blk = jax.lax.broadcasted_iota(jnp.int32, (1, 32), 1)
    raw = jnp.zeros((1, 32), jnp.int32)
    for e in range(E):
        raw = raw + (blk * BT >= cum_pad[0, e].astype(jnp.int32)).astype(jnp.int32)
    nreal = (cum_pad[0, E - 1] / BT).astype(jnp.int32)
    last_cnt = jnp.int32(0)
    for e in range(E):
        last_cnt = last_cnt + (
            cum_pad[0, e].astype(jnp.int32) <= (nreal - 1) * BT).astype(jnp.int32)
    last_e = jnp.minimum(last_cnt, E - 1)
    isreal = (blk < nreal).astype(jnp.int32)
    val = jnp.minimum(raw, E - 1) * isreal + last_e * (1 - isreal)
    isnb = (blk == NB).astype(jnp.int32)
    be_ref[...] = nreal * isnb + val * (1 - isnb)


def _router(x, Wg):
    return pl.pallas_call(
        _router_body,
        out_shape=(
            jax.ShapeDtypeStruct((NR, RB), jnp.int32),
            jax.ShapeDtypeStruct((NR, RB), jnp.float32),
            jax.ShapeDtypeStruct((1, 32), jnp.int32),
            jax.ShapeDtypeStruct((T, D), jnp.bfloat16),
        ),
    )(x, Wg)


W = 16
CPW = NP // W
PW = P // W


def _scatter_body(pos_hbm, w_hbm, st_hbm, sw_hbm,
                  pos0, pos1, w0, w1, tok0, tok1, zb_i, zb_f,
                  st_sh, sw_sh):
    cid = lax.axis_index("c")
    wid = lax.axis_index("s")
    lanes = lax.broadcasted_iota(jnp.int32, (16,), 0)
    base = wid * CPW

    pltpu.sync_copy(pos_hbm.at[pl.ds(base, RB)], pos0)
    pltpu.sync_copy(pos_hbm.at[pl.ds(base + RB, RB)], pos1)
    pltpu.sync_copy(w_hbm.at[pl.ds(base, RB)], w0)
    pltpu.sync_copy(w_hbm.at[pl.ds(base + RB, RB)], w1)

    for c in range(RB // 16):
        tok0[pl.ds(16 * c, 16)] = (base + 16 * c + lanes) & (T - 1)
        tok1[pl.ds(16 * c, 16)] = (base + RB + 16 * c + lanes) & (T - 1)

    for c in range(PW // 16):
        zb_i[pl.ds(16 * c, 16)] = jnp.zeros((16,), jnp.int32)
        zb_f[pl.ds(16 * c, 16)] = jnp.zeros((16,), jnp.float32)
    pltpu.sync_copy(zb_i, st_sh.at[pl.ds(wid * PW, PW)])
    pltpu.sync_copy(zb_f, sw_sh.at[pl.ds(wid * PW, PW)])
    plsc.subcore_barrier()

    pltpu.sync_copy(tok0, st_sh.at[pos0])
    pltpu.sync_copy(tok1, st_sh.at[pos1])
    pltpu.sync_copy(w0, sw_sh.at[pos0])
    pltpu.sync_copy(w1, sw_sh.at[pos1])
    plsc.subcore_barrier()

    @pl.when(cid == 0)
    def _():
        pltpu.sync_copy(st_sh.at[pl.ds(wid * PW, PW)], zb_i)
        pltpu.sync_copy(zb_i, st_hbm.at[pl.ds(wid * PW, PW)])
        pltpu.sync_copy(sw_sh.at[pl.ds(wid * PW, PW)], zb_f)
        pltpu.sync_copy(zb_f, sw_hbm.at[pl.ds(wid * PW, PW)])


@functools.cache
def _make_scatter():
    @functools.partial(
        pl.kernel,
        out_type=(
            jax.ShapeDtypeStruct((P,), jnp.int32),
            jax.ShapeDtypeStruct((P,), jnp.float32),
        ),
        mesh=plsc.VectorSubcoreMesh(core_axis_name="c", subcore_axis_name="s"),
        scratch_types=[
            pltpu.VMEM((RB,), jnp.int32),
            pltpu.VMEM((RB,), jnp.int32),
            pltpu.VMEM((RB,), jnp.float32),
            pltpu.VMEM((RB,), jnp.float32),
            pltpu.VMEM((RB,), jnp.int32),
            pltpu.VMEM((RB,), jnp.int32),
            pltpu.VMEM((PW,), jnp.int32),
            pltpu.VMEM((PW,), jnp.float32),
            pltpu.VMEM_SHARED((P,), jnp.int32),
            pltpu.VMEM_SHARED((P,), jnp.float32),
        ],
    )
    def _scatter_kernel(pos_hbm, w_hbm, st_hbm, sw_hbm, *scratch):
        _scatter_body(pos_hbm, w_hbm, st_hbm, sw_hbm, *scratch)

    return _scatter_kernel


def _scatter(pos, wflat):
    return _make_scatter()(pos.reshape(NP), wflat.reshape(NP))


def _moe_body(be_ref, x_ref, st_ref, wg_ref, wu_ref, wd_ref, ws_ref, out_ref):
    i = pl.program_id(0)

    @pl.when(i == 0)
    def _():
        out_ref[...] = jnp.zeros_like(out_ref)

    @pl.when(i < be_ref[NB])
    def _():
        tok_iota = jax.lax.broadcasted_iota(jnp.int32, (BT, T), 1)
        st_col = st_ref[0, 0].reshape(BT, 1)
        onehot = (tok_iota == st_col).astype(jnp.bfloat16)
        xb = jnp.dot(onehot, x_ref[...],
                     preferred_element_type=jnp.float32
                     ).astype(jnp.bfloat16)
        wg = wg_ref[0].astype(jnp.bfloat16)
        wu = wu_ref[0].astype(jnp.bfloat16)
        wd = wd_ref[0].astype(jnp.bfloat16)
        g = jnp.dot(xb, wg, preferred_element_type=jnp.float32)
        u = jnp.dot(xb, wu, preferred_element_type=jnp.float32)
        h = ((g * jax.nn.sigmoid(g)) * u).astype(jnp.bfloat16)
        y = jnp.dot(h, wd, preferred_element_type=jnp.float32)
        ysc = (y * ws_ref[0, 0].reshape(BT, 1)).astype(jnp.bfloat16)
        out_ref[...] += jax.lax.dot_general(
            onehot, ysc, (((0,), (0,)), ((), ())),
            preferred_element_type=jnp.float32)


def _moe_fused(block_expert, x16, sort_tok, w_gate, w_up, w_down, sort_w):
    grid_spec = pltpu.PrefetchScalarGridSpec(
        num_scalar_prefetch=1,
        grid=(NB,),
        in_specs=[
            pl.BlockSpec((T, D), lambda i, be: (0, 0)),
            pl.BlockSpec((1, 1, BT), lambda i, be: (i, 0, 0)),
            pl.BlockSpec((1, D, F), lambda i, be: (be[i], 0, 0)),
            pl.BlockSpec((1, D, F), lambda i, be: (be[i], 0, 0)),
            pl.BlockSpec((1, F, D), lambda i, be: (be[i], 0, 0)),
            pl.BlockSpec((1, 1, BT), lambda i, be: (i, 0, 0)),
        ],
        out_specs=pl.BlockSpec((T, D), lambda i, be: (0, 0)),
    )
    return pl.pallas_call(
        _moe_body,
        grid_spec=grid_spec,
        out_shape=jax.ShapeDtypeStruct((T, D), jnp.float32),
    )(block_expert, x16, sort_tok.reshape(NB, 1, BT), w_gate, w_up, w_down,
      sort_w.reshape(NB, 1, BT))


def kernel(x, Wg, w_gate, w_up, w_down):
    pos, wflat, be2, x16 = _router(x, Wg)
    sort_tok, sort_w = _scatter(pos, wflat)
    block_expert = be2[0, :NB + 1]

    return _moe_fused(block_expert, x16, sort_tok,
                      w_gate, w_up, w_down, sort_w)
